# baseline (speedup 1.0000x reference)
import functools

import jax
import jax.numpy as jnp
from jax import lax
from jax.experimental import pallas as pl
from jax.experimental.pallas import tpu as pltpu
from jax.experimental.pallas import tpu_sc as plsc

NC = 2
NS = 16
NW = NC * NS
L = 16
RPT = 320
CHF = 64
PCAP = 80
SCE = 512


def _deg_partials_kernel(n_nodes, e_pad):
    ep = e_pad // NW
    mesh = plsc.VectorSubcoreMesh(
        core_axis_name="c", subcore_axis_name="s", num_cores=NC, num_subcores=NS
    )

    @functools.partial(
        pl.kernel,
        mesh=mesh,
        out_type=jax.ShapeDtypeStruct((NW, n_nodes), jnp.float32),
        compiler_params=pltpu.CompilerParams(needs_layout_passes=False),
        scratch_types=[
            pltpu.VMEM((ep,), jnp.int32),
            pltpu.VMEM((ep,), jnp.float32),
            pltpu.VMEM((n_nodes,), jnp.float32),
        ],
    )
    def k(dst_hbm, w_hbm, parts_hbm, dstv, wv, degl):
        c = lax.axis_index("c")
        s = lax.axis_index("s")
        wid = c * NS + s
        pltpu.sync_copy(dst_hbm.at[pl.ds(wid * ep, ep)], dstv)
        pltpu.sync_copy(w_hbm.at[pl.ds(wid * ep, ep)], wv)

        @pl.loop(0, n_nodes // L)
        def _(i):
            degl[pl.ds(i * L, L)] = jnp.zeros((L,), jnp.float32)

        @pl.loop(0, ep // L)
        def _(i):
            idx = dstv[pl.ds(i * L, L)]
            val = wv[pl.ds(i * L, L)]
            plsc.addupdate_scatter(degl, [idx], val)

        pltpu.sync_copy(degl, parts_hbm.at[wid])

    return k


def _dis_kernel(parts):
    def body(p_ref, dis_ref):
        deg = jnp.sum(p_ref[...], axis=0, keepdims=True)
        dis_ref[...] = jnp.where(
            deg > 0, lax.rsqrt(jnp.maximum(deg, 1e-12)), 0.0
        )

    n = parts.shape[1]
    return pl.pallas_call(
        body,
        out_shape=jax.ShapeDtypeStruct((1, n), jnp.float32),
    )(parts)


def _matmul_kernel(x, w):
    n, d_in = x.shape
    d_out = w.shape[1]
    blk = 1000

    def body(x_ref, w_ref, h_ref):
        h_ref[...] = lax.dot_general(
            x_ref[...], w_ref[...],
            (((1,), (0,)), ((), ())),
            precision=lax.Precision.HIGHEST,
            preferred_element_type=jnp.float32,
        )

    return pl.pallas_call(
        body,
        grid=(n // blk,),
        in_specs=[
            pl.BlockSpec((blk, d_in), lambda i: (i, 0)),
            pl.BlockSpec((d_in, d_out), lambda i: (0, 0)),
        ],
        out_specs=pl.BlockSpec((blk, d_out), lambda i: (i, 0)),
        out_shape=jax.ShapeDtypeStruct((n, d_out), jnp.float32),
    )(x, w)


def _aggregate_kernel(n_nodes, d, e_pad):
    npad = NW * RPT
    nsup = e_pad // SCE
    mesh = plsc.VectorSubcoreMesh(
        core_axis_name="c", subcore_axis_name="s", num_cores=NC, num_subcores=NS
    )

    @functools.partial(
        pl.kernel,
        mesh=mesh,
        out_type=jax.ShapeDtypeStruct((npad, d), jnp.float32),
        compiler_params=pltpu.CompilerParams(needs_layout_passes=False),
        scratch_types=[
            pltpu.VMEM((n_nodes,), jnp.float32),
            pltpu.VMEM((2 * SCE,), jnp.int32),
            pltpu.VMEM((2 * SCE,), jnp.int32),
            pltpu.VMEM((2 * SCE,), jnp.float32),
            pltpu.VMEM((PCAP,), jnp.int32),
            pltpu.VMEM((PCAP,), jnp.float32),
            pltpu.VMEM((2 * CHF,), jnp.int32),
            pltpu.VMEM((2 * CHF,), jnp.float32),
            pltpu.VMEM((2 * CHF,), jnp.int32),
            pltpu.VMEM((2 * CHF, d), jnp.float32),
            pltpu.VMEM((RPT, d), jnp.float32),
            pltpu.SemaphoreType.DMA,
            pltpu.SemaphoreType.DMA,
            pltpu.SemaphoreType.DMA,
            pltpu.SemaphoreType.DMA,
        ],
    )
    def k(src_hbm, dst_hbm, w_hbm, dis_hbm, h_hbm, out_hbm,
          disv, scs, scd, scw, ppack, pw, gidx, wmbuf, locbuf, rows,
          acc, sem_s0, sem_s1, sem_f0, sem_f1):
        c = lax.axis_index("c")
        s = lax.axis_index("s")
        wid = c * NS + s
        base = wid * RPT

        pltpu.sync_copy(dis_hbm, disv)

        @pl.loop(0, RPT)
        def _(r):
            for l in range(d // L):
                acc[r, pl.ds(l * L, L)] = jnp.zeros((L,), jnp.float32)

        for j in range(PCAP // L):
            sl = pl.ds(j * L, L)
            ppack[sl] = jnp.zeros((L,), jnp.int32)
            pw[sl] = jnp.zeros((L,), jnp.float32)

        def accumulate(q):
            qo = q * CHF
            lane = lax.iota(jnp.int32, L)

            @pl.loop(0, CHF, unroll=4)
            def _(r):
                rf = jnp.full((L,), qo + r, jnp.int32)
                wbs = plsc.load_gather(wmbuf, [rf])
                lv = plsc.load_gather(locbuf, [rf])
                for l in range(d // L):
                    sl = pl.ds(l * L, L)
                    plsc.addupdate_scatter(
                        acc, [lv, lane + (l * L)], rows[qo + r, sl] * wbs
                    )

        def wait_flush(q):
            if q == 0:
                pltpu.make_async_copy(
                    h_hbm.at[gidx.at[pl.ds(0, CHF)]],
                    rows.at[pl.ds(0, CHF)], sem_f0).wait()
            else:
                pltpu.make_async_copy(
                    h_hbm.at[gidx.at[pl.ds(CHF, CHF)]],
                    rows.at[pl.ds(CHF, CHF)], sem_f1).wait()

        def flush_fire(count, fc):
            p = lax.rem(fc, 2)
            po = p * CHF
            for j in range(CHF // L):
                sl = pl.ds(j * L, L)
                osl = pl.ds(po + j * L, L)
                pp16 = ppack[sl]
                w16 = pw[sl]
                s16 = pp16 & 16383
                loc16 = lax.shift_right_logical(pp16, 14)
                mk = (lax.iota(jnp.int32, L) + (j * L)) < count
                a16 = plsc.load_gather(disv, [s16])
                b16 = plsc.load_gather(disv, [loc16 + base])
                gidx[osl] = s16
                wmbuf[osl] = jnp.where(mk, w16 * a16 * b16, 0.0)
                locbuf[osl] = jnp.where(mk, loc16, 0)

            @pl.when(p == 0)
            def _():
                pltpu.async_copy(h_hbm.at[gidx.at[pl.ds(0, CHF)]],
                                 rows.at[pl.ds(0, CHF)], sem_f0)

            @pl.when(p == 1)
            def _():
                pltpu.async_copy(h_hbm.at[gidx.at[pl.ds(CHF, CHF)]],
                                 rows.at[pl.ds(CHF, CHF)], sem_f1)

            @pl.when((fc >= 1) & (p == 1))
            def _():
                wait_flush(0)
                accumulate(0)

            @pl.when((fc >= 1) & (p == 0))
            def _():
                wait_flush(1)
                accumulate(1)

        u0 = wid * nsup // NW

        def issue_scan(i):
            u = lax.rem(u0 + i, nsup)

            @pl.when(lax.rem(i, 2) == 0)
            def _():
                pltpu.async_copy(src_hbm.at[pl.ds(u * SCE, SCE)],
                                 scs.at[pl.ds(0, SCE)], sem_s0)
                pltpu.async_copy(dst_hbm.at[pl.ds(u * SCE, SCE)],
                                 scd.at[pl.ds(0, SCE)], sem_s0)
                pltpu.async_copy(w_hbm.at[pl.ds(u * SCE, SCE)],
                                 scw.at[pl.ds(0, SCE)], sem_s0)

            @pl.when(lax.rem(i, 2) == 1)
            def _():
                pltpu.async_copy(src_hbm.at[pl.ds(u * SCE, SCE)],
                                 scs.at[pl.ds(SCE, SCE)], sem_s1)
                pltpu.async_copy(dst_hbm.at[pl.ds(u * SCE, SCE)],
                                 scd.at[pl.ds(SCE, SCE)], sem_s1)
                pltpu.async_copy(w_hbm.at[pl.ds(u * SCE, SCE)],
                                 scw.at[pl.ds(SCE, SCE)], sem_s1)

        def wait_scan(b):
            off = b * SCE
            sem = sem_s0 if b == 0 else sem_s1
            pltpu.make_async_copy(src_hbm.at[pl.ds(0, SCE)],
                                  scs.at[pl.ds(off, SCE)], sem).wait()
            pltpu.make_async_copy(dst_hbm.at[pl.ds(0, SCE)],
                                  scd.at[pl.ds(off, SCE)], sem).wait()
            pltpu.make_async_copy(w_hbm.at[pl.ds(0, SCE)],
                                  scw.at[pl.ds(off, SCE)], sem).wait()

        issue_scan(jnp.int32(0))

        @pl.loop(0, nsup, init_carry=(jnp.int32(0), jnp.int32(0)))
        def carry_fin(i, carry0):
            @pl.when(i + 1 < nsup)
            def _():
                issue_scan(i + 1)

            b = lax.rem(i, 2)

            @pl.when(b == 0)
            def _():
                wait_scan(0)

            @pl.when(b == 1)
            def _():
                wait_scan(1)

            off_b = b * SCE

            @pl.loop(0, SCE // L, init_carry=carry0, unroll=4)
            def carry_in(v, carry):
                cnt, fc = carry
                sl = pl.ds(off_b + v * L, L)
                d16 = scd[sl]
                s16 = scs[sl]
                w16 = scw[sl]
                m = (d16 >= base) & (d16 < base + RPT)
                pp16 = s16 | lax.shift_left(d16 - base, 14)
                plsc.store_compressed(ppack.at[pl.ds(cnt, L)], pp16, mask=m)
                plsc.store_compressed(pw.at[pl.ds(cnt, L)], w16, mask=m)
                pc = plsc.all_reduce_population_count(m)[0]
                cnt2 = cnt + pc
                full = cnt2 >= CHF

                @pl.when(full)
                def _():
                    flush_fire(jnp.int32(CHF), fc)
                    ppack[pl.ds(0, L)] = ppack[pl.ds(CHF, L)]
                    pw[pl.ds(0, L)] = pw[pl.ds(CHF, L)]

                return (jnp.where(full, cnt2 - CHF, cnt2),
                        jnp.where(full, fc + 1, fc))

            return carry_in

        cnt_fin, fc_fin = carry_fin
        flush_fire(cnt_fin, fc_fin)

        @pl.when(lax.rem(fc_fin, 2) == 0)
        def _():
            wait_flush(0)
            accumulate(0)

        @pl.when(lax.rem(fc_fin, 2) == 1)
        def _():
            wait_flush(1)
            accumulate(1)

        @pl.loop(0, RPT // 8)
        def _(g):
            pltpu.sync_copy(
                acc.at[pl.ds(g * 8, 8)], out_hbm.at[pl.ds(base + g * 8, 8)]
            )

    return k


def _finish_kernel(acc, b2, a2):
    n, d = acc.shape
    blk = 1000

    def body(acc_ref, b_ref, a_ref, out_ref):
        o = acc_ref[...] + b_ref[...]
        out_ref[...] = jnp.where(o >= 0, o, a_ref[0, 0] * o)

    return pl.pallas_call(
        body,
        grid=(n // blk,),
        in_specs=[
            pl.BlockSpec((blk, d), lambda i: (i, 0)),
            pl.BlockSpec((1, d), lambda i: (0, 0)),
            pl.BlockSpec((1, 1), lambda i: (0, 0), memory_space=pltpu.SMEM),
        ],
        out_specs=pl.BlockSpec((blk, d), lambda i: (i, 0)),
        out_shape=jax.ShapeDtypeStruct((n, d), jnp.float32),
    )(acc, b2, a2)


def kernel(x, edge_index, edge_weight, W, b, prelu_a):
    n, _ = x.shape
    d = W.shape[1]
    e = edge_weight.shape[0]

    ef = e + n
    e_pad = ((ef + SCE - 1) // SCE) * SCE
    loop_idx = jnp.arange(n, dtype=jnp.int32)
    src_f = jnp.concatenate([edge_index[0].astype(jnp.int32), loop_idx])
    dst_f = jnp.concatenate([edge_index[1].astype(jnp.int32), loop_idx])
    w_f = jnp.concatenate([edge_weight, jnp.ones((n,), jnp.float32)])
    pad = e_pad - ef
    pad_idx = jnp.arange(pad, dtype=jnp.int32) % n
    src_f = jnp.concatenate([src_f, pad_idx])
    dst_f = jnp.concatenate([dst_f, pad_idx])
    w_f = jnp.pad(w_f, (0, pad))

    parts = _deg_partials_kernel(n, e_pad)(dst_f, w_f)
    dis = _dis_kernel(parts).reshape((n,))
    h = _matmul_kernel(x, W)

    acc = _aggregate_kernel(n, d, e_pad)(src_f, dst_f, w_f, dis, h)[:n]

    return _finish_kernel(acc, b.reshape(1, d), prelu_a.reshape(1, 1))


# speedup vs baseline: 1.0106x; 1.0106x over previous
import functools

import jax
import jax.numpy as jnp
from jax import lax
from jax.experimental import pallas as pl
from jax.experimental.pallas import tpu as pltpu
from jax.experimental.pallas import tpu_sc as plsc

NC = 2
NS = 16
NW = NC * NS
L = 16
RPT = 320
CHF = 48
PCAP = 64
SCE = 1024


def _deg_partials_kernel(n_nodes, e_pad):
    ep = e_pad // NW
    mesh = plsc.VectorSubcoreMesh(
        core_axis_name="c", subcore_axis_name="s", num_cores=NC, num_subcores=NS
    )

    @functools.partial(
        pl.kernel,
        mesh=mesh,
        out_type=jax.ShapeDtypeStruct((NW, n_nodes), jnp.float32),
        compiler_params=pltpu.CompilerParams(needs_layout_passes=False),
        scratch_types=[
            pltpu.VMEM((ep,), jnp.int32),
            pltpu.VMEM((ep,), jnp.float32),
            pltpu.VMEM((n_nodes,), jnp.float32),
        ],
    )
    def k(dst_hbm, w_hbm, parts_hbm, dstv, wv, degl):
        c = lax.axis_index("c")
        s = lax.axis_index("s")
        wid = c * NS + s
        pltpu.sync_copy(dst_hbm.at[pl.ds(wid * ep, ep)], dstv)
        pltpu.sync_copy(w_hbm.at[pl.ds(wid * ep, ep)], wv)

        @pl.loop(0, n_nodes // L)
        def _(i):
            degl[pl.ds(i * L, L)] = jnp.zeros((L,), jnp.float32)

        @pl.loop(0, ep // L)
        def _(i):
            idx = dstv[pl.ds(i * L, L)]
            val = wv[pl.ds(i * L, L)]
            plsc.addupdate_scatter(degl, [idx], val)

        pltpu.sync_copy(degl, parts_hbm.at[wid])

    return k


def _dis_kernel(parts):
    def body(p_ref, dis_ref):
        deg = jnp.sum(p_ref[...], axis=0, keepdims=True)
        dis_ref[...] = jnp.where(
            deg > 0, lax.rsqrt(jnp.maximum(deg, 1e-12)), 0.0
        )

    n = parts.shape[1]
    return pl.pallas_call(
        body,
        out_shape=jax.ShapeDtypeStruct((1, n), jnp.float32),
    )(parts)


def _matmul_kernel(x, w):
    n, d_in = x.shape
    d_out = w.shape[1]
    blk = 1000

    def body(x_ref, w_ref, h_ref):
        h_ref[...] = lax.dot_general(
            x_ref[...], w_ref[...],
            (((1,), (0,)), ((), ())),
            precision=lax.Precision.HIGHEST,
            preferred_element_type=jnp.float32,
        )

    return pl.pallas_call(
        body,
        grid=(n // blk,),
        in_specs=[
            pl.BlockSpec((blk, d_in), lambda i: (i, 0)),
            pl.BlockSpec((d_in, d_out), lambda i: (0, 0)),
        ],
        out_specs=pl.BlockSpec((blk, d_out), lambda i: (i, 0)),
        out_shape=jax.ShapeDtypeStruct((n, d_out), jnp.float32),
    )(x, w)


def _aggregate_kernel(n_nodes, d, e_pad):
    npad = NW * RPT
    nsup = e_pad // SCE
    mesh = plsc.VectorSubcoreMesh(
        core_axis_name="c", subcore_axis_name="s", num_cores=NC, num_subcores=NS
    )

    @functools.partial(
        pl.kernel,
        mesh=mesh,
        out_type=jax.ShapeDtypeStruct((npad, d), jnp.float32),
        compiler_params=pltpu.CompilerParams(needs_layout_passes=False),
        scratch_types=[
            pltpu.VMEM((n_nodes,), jnp.float32),
            pltpu.VMEM((2 * SCE,), jnp.int32),
            pltpu.VMEM((2 * SCE,), jnp.int32),
            pltpu.VMEM((2 * SCE,), jnp.float32),
            pltpu.VMEM((PCAP,), jnp.int32),
            pltpu.VMEM((PCAP,), jnp.float32),
            pltpu.VMEM((2 * CHF,), jnp.int32),
            pltpu.VMEM((2 * CHF,), jnp.float32),
            pltpu.VMEM((2 * CHF,), jnp.int32),
            pltpu.VMEM((2 * CHF, d), jnp.float32),
            pltpu.VMEM((RPT, d), jnp.float32),
            pltpu.SemaphoreType.DMA,
            pltpu.SemaphoreType.DMA,
            pltpu.SemaphoreType.DMA,
            pltpu.SemaphoreType.DMA,
        ],
    )
    def k(src_hbm, dst_hbm, w_hbm, dis_hbm, h_hbm, out_hbm,
          disv, scs, scd, scw, ppack, pw, gidx, wmbuf, locbuf, rows,
          acc, sem_s0, sem_s1, sem_f0, sem_f1):
        c = lax.axis_index("c")
        s = lax.axis_index("s")
        wid = c * NS + s
        base = wid * RPT

        pltpu.sync_copy(dis_hbm, disv)

        @pl.loop(0, RPT)
        def _(r):
            for l in range(d // L):
                acc[r, pl.ds(l * L, L)] = jnp.zeros((L,), jnp.float32)

        for j in range(PCAP // L):
            sl = pl.ds(j * L, L)
            ppack[sl] = jnp.zeros((L,), jnp.int32)
            pw[sl] = jnp.zeros((L,), jnp.float32)

        def accumulate(q):
            qo = q * CHF
            lane = lax.iota(jnp.int32, L)

            @pl.loop(0, CHF, unroll=4)
            def _(r):
                rf = jnp.full((L,), qo + r, jnp.int32)
                wbs = plsc.load_gather(wmbuf, [rf])
                lv = plsc.load_gather(locbuf, [rf])
                for l in range(d // L):
                    sl = pl.ds(l * L, L)
                    plsc.addupdate_scatter(
                        acc, [lv, lane + (l * L)], rows[qo + r, sl] * wbs
                    )

        def wait_flush(q):
            if q == 0:
                pltpu.make_async_copy(
                    h_hbm.at[gidx.at[pl.ds(0, CHF)]],
                    rows.at[pl.ds(0, CHF)], sem_f0).wait()
            else:
                pltpu.make_async_copy(
                    h_hbm.at[gidx.at[pl.ds(CHF, CHF)]],
                    rows.at[pl.ds(CHF, CHF)], sem_f1).wait()

        def flush_fire(count, fc):
            p = lax.rem(fc, 2)
            po = p * CHF
            for j in range(CHF // L):
                sl = pl.ds(j * L, L)
                osl = pl.ds(po + j * L, L)
                pp16 = ppack[sl]
                w16 = pw[sl]
                s16 = pp16 & 16383
                loc16 = lax.shift_right_logical(pp16, 14)
                mk = (lax.iota(jnp.int32, L) + (j * L)) < count
                a16 = plsc.load_gather(disv, [s16])
                b16 = plsc.load_gather(disv, [loc16 + base])
                gidx[osl] = s16
                wmbuf[osl] = jnp.where(mk, w16 * a16 * b16, 0.0)
                locbuf[osl] = jnp.where(mk, loc16, 0)

            @pl.when(p == 0)
            def _():
                pltpu.async_copy(h_hbm.at[gidx.at[pl.ds(0, CHF)]],
                                 rows.at[pl.ds(0, CHF)], sem_f0)

            @pl.when(p == 1)
            def _():
                pltpu.async_copy(h_hbm.at[gidx.at[pl.ds(CHF, CHF)]],
                                 rows.at[pl.ds(CHF, CHF)], sem_f1)

            @pl.when((fc >= 1) & (p == 1))
            def _():
                wait_flush(0)
                accumulate(0)

            @pl.when((fc >= 1) & (p == 0))
            def _():
                wait_flush(1)
                accumulate(1)

        u0 = wid * nsup // NW

        def issue_scan(i):
            u = lax.rem(u0 + i, nsup)

            @pl.when(lax.rem(i, 2) == 0)
            def _():
                pltpu.async_copy(src_hbm.at[pl.ds(u * SCE, SCE)],
                                 scs.at[pl.ds(0, SCE)], sem_s0)
                pltpu.async_copy(dst_hbm.at[pl.ds(u * SCE, SCE)],
                                 scd.at[pl.ds(0, SCE)], sem_s0)
                pltpu.async_copy(w_hbm.at[pl.ds(u * SCE, SCE)],
                                 scw.at[pl.ds(0, SCE)], sem_s0)

            @pl.when(lax.rem(i, 2) == 1)
            def _():
                pltpu.async_copy(src_hbm.at[pl.ds(u * SCE, SCE)],
                                 scs.at[pl.ds(SCE, SCE)], sem_s1)
                pltpu.async_copy(dst_hbm.at[pl.ds(u * SCE, SCE)],
                                 scd.at[pl.ds(SCE, SCE)], sem_s1)
                pltpu.async_copy(w_hbm.at[pl.ds(u * SCE, SCE)],
                                 scw.at[pl.ds(SCE, SCE)], sem_s1)

        def wait_scan(b):
            off = b * SCE
            sem = sem_s0 if b == 0 else sem_s1
            pltpu.make_async_copy(src_hbm.at[pl.ds(0, SCE)],
                                  scs.at[pl.ds(off, SCE)], sem).wait()
            pltpu.make_async_copy(dst_hbm.at[pl.ds(0, SCE)],
                                  scd.at[pl.ds(off, SCE)], sem).wait()
            pltpu.make_async_copy(w_hbm.at[pl.ds(0, SCE)],
                                  scw.at[pl.ds(off, SCE)], sem).wait()

        issue_scan(jnp.int32(0))

        @pl.loop(0, nsup, init_carry=(jnp.int32(0), jnp.int32(0)))
        def carry_fin(i, carry0):
            @pl.when(i + 1 < nsup)
            def _():
                issue_scan(i + 1)

            b = lax.rem(i, 2)

            @pl.when(b == 0)
            def _():
                wait_scan(0)

            @pl.when(b == 1)
            def _():
                wait_scan(1)

            off_b = b * SCE

            @pl.loop(0, SCE // L, init_carry=carry0, unroll=4)
            def carry_in(v, carry):
                cnt, fc = carry
                sl = pl.ds(off_b + v * L, L)
                d16 = scd[sl]
                s16 = scs[sl]
                w16 = scw[sl]
                m = (d16 >= base) & (d16 < base + RPT)
                pp16 = s16 | lax.shift_left(d16 - base, 14)
                plsc.store_compressed(ppack.at[pl.ds(cnt, L)], pp16, mask=m)
                plsc.store_compressed(pw.at[pl.ds(cnt, L)], w16, mask=m)
                pc = plsc.all_reduce_population_count(m)[0]
                cnt2 = cnt + pc
                full = cnt2 >= CHF

                @pl.when(full)
                def _():
                    flush_fire(jnp.int32(CHF), fc)
                    ppack[pl.ds(0, L)] = ppack[pl.ds(CHF, L)]
                    pw[pl.ds(0, L)] = pw[pl.ds(CHF, L)]

                return (jnp.where(full, cnt2 - CHF, cnt2),
                        jnp.where(full, fc + 1, fc))

            return carry_in

        cnt_fin, fc_fin = carry_fin
        flush_fire(cnt_fin, fc_fin)

        @pl.when(lax.rem(fc_fin, 2) == 0)
        def _():
            wait_flush(0)
            accumulate(0)

        @pl.when(lax.rem(fc_fin, 2) == 1)
        def _():
            wait_flush(1)
            accumulate(1)

        @pl.loop(0, RPT // 8)
        def _(g):
            pltpu.sync_copy(
                acc.at[pl.ds(g * 8, 8)], out_hbm.at[pl.ds(base + g * 8, 8)]
            )

    return k


def _finish_kernel(acc, b2, a2):
    n, d = acc.shape
    blk = 1000

    def body(acc_ref, b_ref, a_ref, out_ref):
        o = acc_ref[...] + b_ref[...]
        out_ref[...] = jnp.where(o >= 0, o, a_ref[0, 0] * o)

    return pl.pallas_call(
        body,
        grid=(n // blk,),
        in_specs=[
            pl.BlockSpec((blk, d), lambda i: (i, 0)),
            pl.BlockSpec((1, d), lambda i: (0, 0)),
            pl.BlockSpec((1, 1), lambda i: (0, 0), memory_space=pltpu.SMEM),
        ],
        out_specs=pl.BlockSpec((blk, d), lambda i: (i, 0)),
        out_shape=jax.ShapeDtypeStruct((n, d), jnp.float32),
    )(acc, b2, a2)


def kernel(x, edge_index, edge_weight, W, b, prelu_a):
    n, _ = x.shape
    d = W.shape[1]
    e = edge_weight.shape[0]

    ef = e + n
    e_pad = ((ef + SCE - 1) // SCE) * SCE
    loop_idx = jnp.arange(n, dtype=jnp.int32)
    src_f = jnp.concatenate([edge_index[0].astype(jnp.int32), loop_idx])
    dst_f = jnp.concatenate([edge_index[1].astype(jnp.int32), loop_idx])
    w_f = jnp.concatenate([edge_weight, jnp.ones((n,), jnp.float32)])
    pad = e_pad - ef
    pad_idx = jnp.arange(pad, dtype=jnp.int32) % n
    src_f = jnp.concatenate([src_f, pad_idx])
    dst_f = jnp.concatenate([dst_f, pad_idx])
    w_f = jnp.pad(w_f, (0, pad))

    parts = _deg_partials_kernel(n, e_pad)(dst_f, w_f)
    dis = _dis_kernel(parts).reshape((n,))
    h = _matmul_kernel(x, W)

    acc = _aggregate_kernel(n, d, e_pad)(src_f, dst_f, w_f, dis, h)[:n]

    return _finish_kernel(acc, b.reshape(1, d), prelu_a.reshape(1, 1))
